# Initial kernel scaffold; baseline (speedup 1.0000x reference)
import jax
import jax.numpy as jnp
from jax.experimental import pallas as pl


def kernel(z_latency, edge_index, operation_id, W_iou, U_iou, b_iou, W_f, U_f, b_f, W_mu, b_mu, W_lv, b_lv, op_table):
    raise NotImplementedError("write your pallas kernel here")


# speedup vs baseline: 3.3111x; 3.3111x over previous
import functools

import jax
import jax.numpy as jnp
from jax import lax
from jax.experimental import pallas as pl
from jax.experimental.pallas import tpu as pltpu
from jax.experimental.pallas import tpu_sc as plsc

_N = 10000
_NP = 10240
_E = 320000
_D = 128
_L = 32
_EB = 64
_NBLK = 314
_EPT = _EB * _NBLK
_EPAD = _EPT * 16
_R = 1024
_WN = _NP // 32
_HB = 32


def _k1_body(z_ref, wiou_ref, wf_ref, uf_ref, biou_ref, bf_ref,
             xiou_ref, g1_ref, g2_ref, g3_ref):
    z = z_ref[...]
    xiou = jnp.dot(z, wiou_ref[...], preferred_element_type=jnp.float32) + biou_ref[...]
    xf = jnp.dot(z, wf_ref[...], preferred_element_type=jnp.float32) + bf_ref[...]
    i = xiou[:, :_D]
    o = xiou[:, _D:2 * _D]
    u = xiou[:, 2 * _D:]
    c1 = jax.nn.sigmoid(i) * jnp.tanh(u)
    h1 = jax.nn.sigmoid(o) * jnp.tanh(c1)
    huf = jnp.dot(h1, uf_ref[...], preferred_element_type=jnp.float32)
    rid = pl.program_id(0) * _R + lax.broadcasted_iota(jnp.int32, (_R, 1), 0)
    m = rid < _N
    h1 = jnp.where(m, h1, 0.0)
    c1 = jnp.where(m, c1, 0.0)
    huf = jnp.where(m, huf, 0.0)
    xiou_ref[...] = xiou
    g1_ref[0] = h1
    g1_ref[1] = c1
    g2_ref[0] = jnp.zeros((_R, _D), jnp.float32)
    g2_ref[1] = huf
    g3_ref[0] = jnp.full((_R, _D), 88.0, jnp.float32)
    g3_ref[1] = xf


_stk_spec = pl.BlockSpec((2, _R, _D), lambda i: (0, i, 0))

_k1 = pl.pallas_call(
    _k1_body,
    grid=(_NP // _R,),
    in_specs=[
        pl.BlockSpec((_R, _D), lambda i: (i, 0)),
        pl.BlockSpec((_D, 3 * _D), lambda i: (0, 0)),
        pl.BlockSpec((_D, _D), lambda i: (0, 0)),
        pl.BlockSpec((_D, _D), lambda i: (0, 0)),
        pl.BlockSpec((1, 3 * _D), lambda i: (0, 0)),
        pl.BlockSpec((1, _D), lambda i: (0, 0)),
    ],
    out_specs=[
        pl.BlockSpec((_R, 3 * _D), lambda i: (i, 0)),
        _stk_spec, _stk_spec, _stk_spec,
    ],
    out_shape=[
        jax.ShapeDtypeStruct((_NP, 3 * _D), jnp.float32),
        jax.ShapeDtypeStruct((2, _NP, _D), jnp.float32),
        jax.ShapeDtypeStruct((2, _NP, _D), jnp.float32),
        jax.ShapeDtypeStruct((2, _NP, _D), jnp.float32),
    ],
)


def _k2_body(xiou_ref, hs_ref, cs_ref, uiou_ref, wmu_ref, wlv_ref,
             bmu_ref, blv_ref, out_ref):
    iou = xiou_ref[...] + jnp.dot(hs_ref[...], uiou_ref[...],
                                  preferred_element_type=jnp.float32)
    i = iou[:, :_D]
    o = iou[:, _D:2 * _D]
    u = iou[:, 2 * _D:]
    c2 = jax.nn.sigmoid(i) * jnp.tanh(u) + cs_ref[...]
    h2 = jax.nn.sigmoid(o) * jnp.tanh(c2)
    mu = jnp.dot(h2, wmu_ref[...], preferred_element_type=jnp.float32) + bmu_ref[...]
    lv = jnp.dot(h2, wlv_ref[...], preferred_element_type=jnp.float32) + blv_ref[...]
    out_ref[...] = jnp.concatenate([mu, lv], axis=-1)


_k2 = pl.pallas_call(
    _k2_body,
    grid=(_NP // _R,),
    in_specs=[
        pl.BlockSpec((_R, 3 * _D), lambda i: (i, 0)),
        pl.BlockSpec((_R, _D), lambda i: (i, 0)),
        pl.BlockSpec((_R, _D), lambda i: (i, 0)),
        pl.BlockSpec((_D, 3 * _D), lambda i: (0, 0)),
        pl.BlockSpec((_D, _L), lambda i: (0, 0)),
        pl.BlockSpec((_D, _L), lambda i: (0, 0)),
        pl.BlockSpec((1, _L), lambda i: (0, 0)),
        pl.BlockSpec((1, _L), lambda i: (0, 0)),
    ],
    out_specs=pl.BlockSpec((_R, 2 * _L), lambda i: (i, 0)),
    out_shape=jax.ShapeDtypeStruct((_NP, 2 * _L), jnp.float32),
)


_mesh = plsc.VectorSubcoreMesh(core_axis_name="c", subcore_axis_name="s")


@functools.partial(
    pl.kernel,
    out_type=jax.ShapeDtypeStruct((2, _NP, _D), jnp.float32),
    mesh=_mesh,
    scratch_types=[
        pltpu.VMEM((_EB,), jnp.int32),
        pltpu.VMEM((_EB,), jnp.int32),
        pltpu.VMEM((_EB,), jnp.int32),
        pltpu.VMEM((_EB, _D), jnp.float32),
        pltpu.VMEM((_EB, _D), jnp.float32),
        pltpu.VMEM((_EB, _D), jnp.float32),
        pltpu.VMEM_SHARED((_NP, _D), jnp.float32),
        pltpu.SemaphoreType.DMA,
    ],
)
def _edge_kernel(g1_hbm, g2_hbm, g3_hbm, src2_hbm, dst2_hbm, dstraw_hbm,
                 out_hbm,
                 sadj, dadj, dstb, g1b, g2b, g3b, acc, sem):
    cid = lax.axis_index("c")
    sid = lax.axis_index("s")
    zeros16 = jnp.zeros((16,), jnp.float32)

    def zrow(r, carry):
        for c in range(_D // 16):
            g1b[r, pl.ds(c * 16, 16)] = zeros16
        return carry

    lax.fori_loop(0, _EB, zrow, 0)
    for k in range(640 // _EB):
        pltpu.sync_copy(g1b, acc.at[pl.ds(sid * 640 + k * _EB, _EB)])
    plsc.subcore_barrier()

    def blk(b, carry):
        off = sid * _EPT + b * _EB
        pltpu.sync_copy(src2_hbm.at[cid, pl.ds(off, _EB)], sadj)
        pltpu.sync_copy(dst2_hbm.at[cid, pl.ds(off, _EB)], dadj)
        pltpu.sync_copy(dstraw_hbm.at[pl.ds(off, _EB)], dstb)
        cp1 = pltpu.async_copy(g1_hbm.at[sadj], g1b, sem)
        cp2 = pltpu.async_copy(g2_hbm.at[sadj], g2b, sem)
        cp3 = pltpu.async_copy(g3_hbm.at[dadj], g3b, sem)
        cp1.wait()
        cp2.wait()
        cp3.wait()

        def crow(r, inner):
            for c in range(_D // 16):
                sl = pl.ds(c * 16, 16)
                t = g3b[r, sl] + g2b[r, sl]
                f = 1.0 / (1.0 + jnp.exp(-t))
                g3b[r, sl] = f * g1b[r, sl]
            return inner

        lax.fori_loop(0, _EB, crow, 0)
        pltpu.sync_copy(g3b, acc.at[dstb], add=True)
        return carry

    lax.fori_loop(0, _NBLK, blk, 0)
    plsc.subcore_barrier()
    r0 = sid * 640
    pltpu.sync_copy(acc.at[pl.ds(r0, 640)], out_hbm.at[cid, pl.ds(r0, 640)])


@functools.partial(
    pl.kernel,
    out_type=jax.ShapeDtypeStruct((_NP, 2 * _L), jnp.float32),
    mesh=_mesh,
    scratch_types=[
        pltpu.VMEM((_HB,), jnp.int32),
        pltpu.VMEM((_HB, 2 * _L * _L), jnp.float32),
        pltpu.VMEM((_HB, 2 * _L), jnp.float32),
        pltpu.VMEM((_HB, 2 * _L), jnp.float32),
        pltpu.SemaphoreType.DMA,
    ],
)
def _head_kernel(opid_hbm, mupre_hbm, table_hbm, out_hbm,
                 idxb, wbuf, mub, outv, sem):
    cid = lax.axis_index("c")
    sid = lax.axis_index("s")
    wid = sid * 2 + cid
    nb = wid * _WN

    def blk(b, carry):
        nb2 = nb + b * _HB
        pltpu.sync_copy(opid_hbm.at[pl.ds(nb2, _HB)], idxb)
        cpw = pltpu.async_copy(table_hbm.at[idxb], wbuf, sem)
        pltpu.sync_copy(mupre_hbm.at[pl.ds(nb2, _HB)], mub)
        cpw.wait()

        def node(n, inner):
            mu0 = jnp.zeros((16,), jnp.float32)
            mu1 = jnp.zeros((16,), jnp.float32)
            lv0 = jnp.zeros((16,), jnp.float32)
            lv1 = jnp.zeros((16,), jnp.float32)
            for lc in range(_L // 16):
                mvec = mub[n, pl.ds(lc * 16, 16)]
                lvec = mub[n, pl.ds(_L + lc * 16, 16)]
                for t in range(16):
                    l = lc * 16 + t
                    mval = mvec[t]
                    lval = lvec[t]
                    base = l * 2 * _L
                    mu0 = mu0 + mval * wbuf[n, pl.ds(base, 16)]
                    mu1 = mu1 + mval * wbuf[n, pl.ds(base + 16, 16)]
                    lv0 = lv0 + lval * wbuf[n, pl.ds(base + 32, 16)]
                    lv1 = lv1 + lval * wbuf[n, pl.ds(base + 48, 16)]
            outv[n, pl.ds(0, 16)] = mu0
            outv[n, pl.ds(16, 16)] = mu1
            outv[n, pl.ds(32, 16)] = lv0
            outv[n, pl.ds(48, 16)] = lv1
            return inner

        lax.fori_loop(0, _HB, node, 0)
        pltpu.sync_copy(outv, out_hbm.at[pl.ds(nb2, _HB)])
        return carry

    lax.fori_loop(0, _WN // _HB, blk, 0)


@jax.jit
def kernel(z_latency, edge_index, operation_id, W_iou, U_iou, b_iou,
           W_f, U_f, b_f, W_mu, b_mu, W_lv, b_lv, op_table):
    z_pad = jnp.pad(z_latency, ((0, _NP - _N), (0, 0)))
    xiou, g1_st, g2_st, g3_st = _k1(
        z_pad, W_iou, W_f, U_f,
        b_iou.reshape(1, 3 * _D), b_f.reshape(1, _D))

    src = edge_index[0]
    dst = edge_index[1]
    pad_e = _EPAD - _E
    src_p = jnp.concatenate([src, jnp.full((pad_e,), _N, jnp.int32)])
    dst_p = jnp.concatenate([dst, jnp.zeros((pad_e,), jnp.int32)])
    src2 = jnp.stack([src_p, src_p + _NP])
    dst2 = jnp.stack([dst_p, dst_p + _NP])

    sums = _edge_kernel(
        g1_st.reshape(2 * _NP, _D), g2_st.reshape(2 * _NP, _D),
        g3_st.reshape(2 * _NP, _D), src2, dst2, dst_p)
    hsum = sums[0]
    csum = sums[1]

    mupre = _k2(xiou, hsum, csum, U_iou, W_mu, W_lv,
                b_mu.reshape(1, _L), b_lv.reshape(1, _L))

    opid_p = jnp.pad(operation_id, (0, _NP - _N))
    head = _head_kernel(opid_p, mupre, op_table)
    return head[:_N, :_L], head[:_N, _L:]


# speedup vs baseline: 4.0499x; 1.2231x over previous
import functools

import jax
import jax.numpy as jnp
from jax import lax
from jax.experimental import pallas as pl
from jax.experimental.pallas import tpu as pltpu
from jax.experimental.pallas import tpu_sc as plsc

_N = 10000
_NP = 10240
_E = 320000
_D = 128
_H = _D // 2
_L = 32
_EB = 64
_NBLK = 314
_EPT = _EB * _NBLK
_EPAD = _EPT * 16
_R = 1024
_WN = _NP // 32
_HB = 32


def _k1_body(z_ref, wiou_ref, wf_ref, uf_ref, biou_ref, bf_ref,
             xiou_ref, a_ref, b_ref):
    z = z_ref[...]
    xiou = jnp.dot(z, wiou_ref[...], preferred_element_type=jnp.float32) + biou_ref[...]
    xf = jnp.dot(z, wf_ref[...], preferred_element_type=jnp.float32) + bf_ref[...]
    i = xiou[:, :_D]
    o = xiou[:, _D:2 * _D]
    u = xiou[:, 2 * _D:]
    c1 = jax.nn.sigmoid(i) * jnp.tanh(u)
    h1 = jax.nn.sigmoid(o) * jnp.tanh(c1)
    huf = jnp.dot(h1, uf_ref[...], preferred_element_type=jnp.float32)
    rid = pl.program_id(0) * _R + lax.broadcasted_iota(jnp.int32, (_R, 1), 0)
    m = rid < _N
    h1 = jnp.where(m, h1, 0.0)
    c1 = jnp.where(m, c1, 0.0)
    huf = jnp.where(m, huf, 0.0)
    xiou_ref[...] = xiou
    a_ref[0] = jnp.concatenate([h1[:, :_H], c1[:, :_H]], axis=1)
    a_ref[1] = jnp.concatenate([h1[:, _H:], c1[:, _H:]], axis=1)
    b_ref[0] = jnp.concatenate([huf[:, :_H], xf[:, :_H]], axis=1)
    b_ref[1] = jnp.concatenate([huf[:, _H:], xf[:, _H:]], axis=1)


_k1 = pl.pallas_call(
    _k1_body,
    grid=(_NP // _R,),
    in_specs=[
        pl.BlockSpec((_R, _D), lambda i: (i, 0)),
        pl.BlockSpec((_D, 3 * _D), lambda i: (0, 0)),
        pl.BlockSpec((_D, _D), lambda i: (0, 0)),
        pl.BlockSpec((_D, _D), lambda i: (0, 0)),
        pl.BlockSpec((1, 3 * _D), lambda i: (0, 0)),
        pl.BlockSpec((1, _D), lambda i: (0, 0)),
    ],
    out_specs=[
        pl.BlockSpec((_R, 3 * _D), lambda i: (i, 0)),
        pl.BlockSpec((2, _R, _D), lambda i: (0, i, 0)),
        pl.BlockSpec((2, _R, _D), lambda i: (0, i, 0)),
    ],
    out_shape=[
        jax.ShapeDtypeStruct((_NP, 3 * _D), jnp.float32),
        jax.ShapeDtypeStruct((2, _NP, _D), jnp.float32),
        jax.ShapeDtypeStruct((2, _NP, _D), jnp.float32),
    ],
)


def _k2_body(xiou_ref, sums_ref, uiou_ref, wmu_ref, wlv_ref,
             bmu_ref, blv_ref, out_ref):
    hs = jnp.concatenate([sums_ref[0, :, :_H], sums_ref[1, :, :_H]], axis=1)
    cs = jnp.concatenate([sums_ref[0, :, _H:], sums_ref[1, :, _H:]], axis=1)
    iou = xiou_ref[...] + jnp.dot(hs, uiou_ref[...],
                                  preferred_element_type=jnp.float32)
    i = iou[:, :_D]
    o = iou[:, _D:2 * _D]
    u = iou[:, 2 * _D:]
    c2 = jax.nn.sigmoid(i) * jnp.tanh(u) + cs
    h2 = jax.nn.sigmoid(o) * jnp.tanh(c2)
    mu = jnp.dot(h2, wmu_ref[...], preferred_element_type=jnp.float32) + bmu_ref[...]
    lv = jnp.dot(h2, wlv_ref[...], preferred_element_type=jnp.float32) + blv_ref[...]
    out_ref[...] = jnp.concatenate([mu, lv], axis=-1)


_k2 = pl.pallas_call(
    _k2_body,
    grid=(_NP // _R,),
    in_specs=[
        pl.BlockSpec((_R, 3 * _D), lambda i: (i, 0)),
        pl.BlockSpec((2, _R, _D), lambda i: (0, i, 0)),
        pl.BlockSpec((_D, 3 * _D), lambda i: (0, 0)),
        pl.BlockSpec((_D, _L), lambda i: (0, 0)),
        pl.BlockSpec((_D, _L), lambda i: (0, 0)),
        pl.BlockSpec((1, _L), lambda i: (0, 0)),
        pl.BlockSpec((1, _L), lambda i: (0, 0)),
    ],
    out_specs=pl.BlockSpec((_R, 2 * _L), lambda i: (i, 0)),
    out_shape=jax.ShapeDtypeStruct((_NP, 2 * _L), jnp.float32),
)


_mesh = plsc.VectorSubcoreMesh(core_axis_name="c", subcore_axis_name="s")


@functools.partial(
    pl.kernel,
    out_type=jax.ShapeDtypeStruct((2, _NP, _D), jnp.float32),
    mesh=_mesh,
    scratch_types=[
        pltpu.VMEM((_EB,), jnp.int32),
        pltpu.VMEM((_EB,), jnp.int32),
        pltpu.VMEM((_EB,), jnp.int32),
        pltpu.VMEM((_EB, _D), jnp.float32),
        pltpu.VMEM((_EB, _D), jnp.float32),
        pltpu.VMEM((_EB, _D), jnp.float32),
        pltpu.VMEM_SHARED((_NP, _D), jnp.float32),
        pltpu.SemaphoreType.DMA,
    ],
)
def _edge_kernel(a_hbm, b_hbm, src2_hbm, dst2_hbm, dstraw_hbm,
                 out_hbm,
                 sadj, dadj, dstb, ab, bsb, bdb, acc, sem):
    cid = lax.axis_index("c")
    sid = lax.axis_index("s")
    zeros16 = jnp.zeros((16,), jnp.float32)

    def zrow(r, carry):
        for c in range(_D // 16):
            ab[r, pl.ds(c * 16, 16)] = zeros16
        return carry

    lax.fori_loop(0, _EB, zrow, 0)
    for k in range(640 // _EB):
        pltpu.sync_copy(ab, acc.at[pl.ds(sid * 640 + k * _EB, _EB)])
    plsc.subcore_barrier()

    def blk(b, carry):
        off = sid * _EPT + b * _EB
        pltpu.sync_copy(src2_hbm.at[cid, pl.ds(off, _EB)], sadj)
        pltpu.sync_copy(dst2_hbm.at[cid, pl.ds(off, _EB)], dadj)
        pltpu.sync_copy(dstraw_hbm.at[pl.ds(off, _EB)], dstb)
        cp1 = pltpu.async_copy(a_hbm.at[sadj], ab, sem)
        cp2 = pltpu.async_copy(b_hbm.at[sadj], bsb, sem)
        cp3 = pltpu.async_copy(b_hbm.at[dadj], bdb, sem)
        cp1.wait()
        cp2.wait()
        cp3.wait()

        def crow(r, inner):
            for c in range(_H // 16):
                sl = pl.ds(c * 16, 16)
                sl2 = pl.ds(_H + c * 16, 16)
                t = bdb[r, sl2] + bsb[r, sl]
                f = 1.0 / (1.0 + jnp.exp(-t))
                ab[r, sl2] = f * ab[r, sl2]
            return inner

        lax.fori_loop(0, _EB, crow, 0)
        pltpu.sync_copy(ab, acc.at[dstb], add=True)
        return carry

    lax.fori_loop(0, _NBLK, blk, 0)
    plsc.subcore_barrier()
    r0 = sid * 640
    pltpu.sync_copy(acc.at[pl.ds(r0, 640)], out_hbm.at[cid, pl.ds(r0, 640)])


@functools.partial(
    pl.kernel,
    out_type=jax.ShapeDtypeStruct((_NP, 2 * _L), jnp.float32),
    mesh=_mesh,
    scratch_types=[
        pltpu.VMEM((_HB,), jnp.int32),
        pltpu.VMEM((_HB, 2 * _L * _L), jnp.float32),
        pltpu.VMEM((_HB, 2 * _L), jnp.float32),
        pltpu.VMEM((_HB, 2 * _L), jnp.float32),
        pltpu.SemaphoreType.DMA,
    ],
)
def _head_kernel(opid_hbm, mupre_hbm, table_hbm, out_hbm,
                 idxb, wbuf, mub, outv, sem):
    cid = lax.axis_index("c")
    sid = lax.axis_index("s")
    wid = sid * 2 + cid
    nb = wid * _WN

    def blk(b, carry):
        nb2 = nb + b * _HB
        pltpu.sync_copy(opid_hbm.at[pl.ds(nb2, _HB)], idxb)
        cpw = pltpu.async_copy(table_hbm.at[idxb], wbuf, sem)
        pltpu.sync_copy(mupre_hbm.at[pl.ds(nb2, _HB)], mub)
        cpw.wait()

        def node(n, inner):
            mu0 = jnp.zeros((16,), jnp.float32)
            mu1 = jnp.zeros((16,), jnp.float32)
            lv0 = jnp.zeros((16,), jnp.float32)
            lv1 = jnp.zeros((16,), jnp.float32)
            for lc in range(_L // 16):
                mvec = mub[n, pl.ds(lc * 16, 16)]
                lvec = mub[n, pl.ds(_L + lc * 16, 16)]
                for t in range(16):
                    l = lc * 16 + t
                    mval = mvec[t]
                    lval = lvec[t]
                    base = l * 2 * _L
                    mu0 = mu0 + mval * wbuf[n, pl.ds(base, 16)]
                    mu1 = mu1 + mval * wbuf[n, pl.ds(base + 16, 16)]
                    lv0 = lv0 + lval * wbuf[n, pl.ds(base + 32, 16)]
                    lv1 = lv1 + lval * wbuf[n, pl.ds(base + 48, 16)]
            outv[n, pl.ds(0, 16)] = mu0
            outv[n, pl.ds(16, 16)] = mu1
            outv[n, pl.ds(32, 16)] = lv0
            outv[n, pl.ds(48, 16)] = lv1
            return inner

        lax.fori_loop(0, _HB, node, 0)
        pltpu.sync_copy(outv, out_hbm.at[pl.ds(nb2, _HB)])
        return carry

    lax.fori_loop(0, _WN // _HB, blk, 0)


@jax.jit
def kernel(z_latency, edge_index, operation_id, W_iou, U_iou, b_iou,
           W_f, U_f, b_f, W_mu, b_mu, W_lv, b_lv, op_table):
    z_pad = jnp.pad(z_latency, ((0, _NP - _N), (0, 0)))
    xiou, a_st, b_st = _k1(
        z_pad, W_iou, W_f, U_f,
        b_iou.reshape(1, 3 * _D), b_f.reshape(1, _D))

    src = edge_index[0]
    dst = edge_index[1]
    pad_e = _EPAD - _E
    src_p = jnp.concatenate([src, jnp.full((pad_e,), _N, jnp.int32)])
    dst_p = jnp.concatenate([dst, jnp.zeros((pad_e,), jnp.int32)])
    src2 = jnp.stack([src_p, src_p + _NP])
    dst2 = jnp.stack([dst_p, dst_p + _NP])

    sums = _edge_kernel(
        a_st.reshape(2 * _NP, _D), b_st.reshape(2 * _NP, _D),
        src2, dst2, dst_p)

    mupre = _k2(xiou, sums, U_iou, W_mu, W_lv,
                b_mu.reshape(1, _L), b_lv.reshape(1, _L))

    opid_p = jnp.pad(operation_id, (0, _NP - _N))
    head = _head_kernel(opid_p, mupre, op_table)
    return head[:_N, :_L], head[:_N, _L:]


# speedup vs baseline: 4.6093x; 1.1381x over previous
import functools

import jax
import jax.numpy as jnp
from jax import lax
from jax.experimental import pallas as pl
from jax.experimental.pallas import tpu as pltpu
from jax.experimental.pallas import tpu_sc as plsc

_N = 10000
_NP = 10240
_E = 320000
_D = 128
_H = _D // 2
_L = 32
_EB = 64
_NBLK = 314
_EPT = _EB * _NBLK
_EPAD = _EPT * 16
_R = 1024
_WN = _NP // 32
_HB = 32


def _k1_body(z_ref, wiou_ref, wf_ref, uf_ref, biou_ref, bf_ref,
             xiou_ref, a_ref, b_ref):
    z = z_ref[...]
    xiou = jnp.dot(z, wiou_ref[...], preferred_element_type=jnp.float32) + biou_ref[...]
    xf = jnp.dot(z, wf_ref[...], preferred_element_type=jnp.float32) + bf_ref[...]
    i = xiou[:, :_D]
    o = xiou[:, _D:2 * _D]
    u = xiou[:, 2 * _D:]
    c1 = jax.nn.sigmoid(i) * jnp.tanh(u)
    h1 = jax.nn.sigmoid(o) * jnp.tanh(c1)
    huf = jnp.dot(h1, uf_ref[...], preferred_element_type=jnp.float32)
    rid = pl.program_id(0) * _R + lax.broadcasted_iota(jnp.int32, (_R, 1), 0)
    m = rid < _N
    h1 = jnp.where(m, h1, 0.0)
    c1 = jnp.where(m, c1, 0.0)
    huf = jnp.where(m, huf, 0.0)
    xiou_ref[...] = xiou
    a_ref[0] = jnp.concatenate([h1[:, :_H], c1[:, :_H]], axis=1)
    a_ref[1] = jnp.concatenate([h1[:, _H:], c1[:, _H:]], axis=1)
    b_ref[0] = jnp.concatenate([huf[:, :_H], xf[:, :_H]], axis=1)
    b_ref[1] = jnp.concatenate([huf[:, _H:], xf[:, _H:]], axis=1)


_k1 = pl.pallas_call(
    _k1_body,
    grid=(_NP // _R,),
    in_specs=[
        pl.BlockSpec((_R, _D), lambda i: (i, 0)),
        pl.BlockSpec((_D, 3 * _D), lambda i: (0, 0)),
        pl.BlockSpec((_D, _D), lambda i: (0, 0)),
        pl.BlockSpec((_D, _D), lambda i: (0, 0)),
        pl.BlockSpec((1, 3 * _D), lambda i: (0, 0)),
        pl.BlockSpec((1, _D), lambda i: (0, 0)),
    ],
    out_specs=[
        pl.BlockSpec((_R, 3 * _D), lambda i: (i, 0)),
        pl.BlockSpec((2, _R, _D), lambda i: (0, i, 0)),
        pl.BlockSpec((2, _R, _D), lambda i: (0, i, 0)),
    ],
    out_shape=[
        jax.ShapeDtypeStruct((_NP, 3 * _D), jnp.float32),
        jax.ShapeDtypeStruct((2, _NP, _D), jnp.float32),
        jax.ShapeDtypeStruct((2, _NP, _D), jnp.float32),
    ],
)


def _k2_body(xiou_ref, sums_ref, uiou_ref, wmu_ref, wlv_ref,
             bmu_ref, blv_ref, out_ref):
    hs = jnp.concatenate([sums_ref[0, :, :_H], sums_ref[1, :, :_H]], axis=1)
    cs = jnp.concatenate([sums_ref[0, :, _H:], sums_ref[1, :, _H:]], axis=1)
    iou = xiou_ref[...] + jnp.dot(hs, uiou_ref[...],
                                  preferred_element_type=jnp.float32)
    i = iou[:, :_D]
    o = iou[:, _D:2 * _D]
    u = iou[:, 2 * _D:]
    c2 = jax.nn.sigmoid(i) * jnp.tanh(u) + cs
    h2 = jax.nn.sigmoid(o) * jnp.tanh(c2)
    mu = jnp.dot(h2, wmu_ref[...], preferred_element_type=jnp.float32) + bmu_ref[...]
    lv = jnp.dot(h2, wlv_ref[...], preferred_element_type=jnp.float32) + blv_ref[...]
    out_ref[...] = jnp.concatenate([mu, lv], axis=-1)


_k2 = pl.pallas_call(
    _k2_body,
    grid=(_NP // _R,),
    in_specs=[
        pl.BlockSpec((_R, 3 * _D), lambda i: (i, 0)),
        pl.BlockSpec((2, _R, _D), lambda i: (0, i, 0)),
        pl.BlockSpec((_D, 3 * _D), lambda i: (0, 0)),
        pl.BlockSpec((_D, _L), lambda i: (0, 0)),
        pl.BlockSpec((_D, _L), lambda i: (0, 0)),
        pl.BlockSpec((1, _L), lambda i: (0, 0)),
        pl.BlockSpec((1, _L), lambda i: (0, 0)),
    ],
    out_specs=pl.BlockSpec((_R, 2 * _L), lambda i: (i, 0)),
    out_shape=jax.ShapeDtypeStruct((_NP, 2 * _L), jnp.float32),
)


_mesh = plsc.VectorSubcoreMesh(core_axis_name="c", subcore_axis_name="s")


@functools.partial(
    pl.kernel,
    out_type=jax.ShapeDtypeStruct((2, _NP, _D), jnp.float32),
    mesh=_mesh,
    scratch_types=[
        pltpu.VMEM((3, _EB), jnp.int32),
        pltpu.VMEM((_EB, _D), jnp.float32),
        pltpu.VMEM((_EB, _D), jnp.float32),
        pltpu.VMEM((_EB, _D), jnp.float32),
        pltpu.VMEM_SHARED((_NP, _D), jnp.float32),
        pltpu.SemaphoreType.DMA,
    ],
)
def _edge_kernel(a_hbm, b_hbm, idx_hbm,
                 out_hbm,
                 ibuf, ab, bsb, bdb, acc, sem):
    cid = lax.axis_index("c")
    sid = lax.axis_index("s")
    zeros16 = jnp.zeros((16,), jnp.float32)

    def zrow(r, carry):
        for c in range(_D // 16):
            ab[r, pl.ds(c * 16, 16)] = zeros16
        return carry

    lax.fori_loop(0, _EB, zrow, 0)
    for k in range(640 // _EB):
        pltpu.sync_copy(ab, acc.at[pl.ds(sid * 640 + k * _EB, _EB)])
    plsc.subcore_barrier()

    def blk(b, carry):
        blkid = sid * _NBLK + b
        pltpu.sync_copy(idx_hbm.at[cid, blkid], ibuf)
        cp1 = pltpu.async_copy(a_hbm.at[ibuf.at[0]], ab, sem)
        cp2 = pltpu.async_copy(b_hbm.at[ibuf.at[0]], bsb, sem)
        cp3 = pltpu.async_copy(b_hbm.at[ibuf.at[1]], bdb, sem)
        cp1.wait()
        cp2.wait()
        cp3.wait()

        def crow(r, inner):
            for c in range(_H // 16):
                sl = pl.ds(c * 16, 16)
                sl2 = pl.ds(_H + c * 16, 16)
                t = bdb[r, sl2] + bsb[r, sl]
                f = 1.0 / (1.0 + jnp.exp(-t))
                ab[r, sl2] = f * ab[r, sl2]
            return inner

        lax.fori_loop(0, _EB, crow, 0)
        pltpu.sync_copy(ab, acc.at[ibuf.at[2]], add=True)
        return carry

    lax.fori_loop(0, _NBLK, blk, 0)
    plsc.subcore_barrier()
    r0 = sid * 640
    pltpu.sync_copy(acc.at[pl.ds(r0, 640)], out_hbm.at[cid, pl.ds(r0, 640)])


@functools.partial(
    pl.kernel,
    out_type=jax.ShapeDtypeStruct((_NP, 2 * _L), jnp.float32),
    mesh=_mesh,
    scratch_types=[
        pltpu.VMEM((_HB,), jnp.int32),
        pltpu.VMEM((_HB, 2 * _L * _L), jnp.float32),
        pltpu.VMEM((_HB, 2 * _L), jnp.float32),
        pltpu.VMEM((_HB, 2 * _L), jnp.float32),
        pltpu.SemaphoreType.DMA,
    ],
)
def _head_kernel(opid_hbm, mupre_hbm, table_hbm, out_hbm,
                 idxb, wbuf, mub, outv, sem):
    cid = lax.axis_index("c")
    sid = lax.axis_index("s")
    wid = sid * 2 + cid
    nb = wid * _WN

    def blk(b, carry):
        nb2 = nb + b * _HB
        pltpu.sync_copy(opid_hbm.at[pl.ds(nb2, _HB)], idxb)
        cpw = pltpu.async_copy(table_hbm.at[idxb], wbuf, sem)
        pltpu.sync_copy(mupre_hbm.at[pl.ds(nb2, _HB)], mub)
        cpw.wait()

        def node(n, inner):
            mu0 = jnp.zeros((16,), jnp.float32)
            mu1 = jnp.zeros((16,), jnp.float32)
            lv0 = jnp.zeros((16,), jnp.float32)
            lv1 = jnp.zeros((16,), jnp.float32)
            for lc in range(_L // 16):
                mvec = mub[n, pl.ds(lc * 16, 16)]
                lvec = mub[n, pl.ds(_L + lc * 16, 16)]
                for t in range(16):
                    l = lc * 16 + t
                    mval = mvec[t]
                    lval = lvec[t]
                    base = l * 2 * _L
                    mu0 = mu0 + mval * wbuf[n, pl.ds(base, 16)]
                    mu1 = mu1 + mval * wbuf[n, pl.ds(base + 16, 16)]
                    lv0 = lv0 + lval * wbuf[n, pl.ds(base + 32, 16)]
                    lv1 = lv1 + lval * wbuf[n, pl.ds(base + 48, 16)]
            outv[n, pl.ds(0, 16)] = mu0
            outv[n, pl.ds(16, 16)] = mu1
            outv[n, pl.ds(32, 16)] = lv0
            outv[n, pl.ds(48, 16)] = lv1
            return inner

        lax.fori_loop(0, _HB, node, 0)
        pltpu.sync_copy(outv, out_hbm.at[pl.ds(nb2, _HB)])
        return carry

    lax.fori_loop(0, _WN // _HB, blk, 0)


@jax.jit
def kernel(z_latency, edge_index, operation_id, W_iou, U_iou, b_iou,
           W_f, U_f, b_f, W_mu, b_mu, W_lv, b_lv, op_table):
    z_pad = jnp.pad(z_latency, ((0, _NP - _N), (0, 0)))
    xiou, a_st, b_st = _k1(
        z_pad, W_iou, W_f, U_f,
        b_iou.reshape(1, 3 * _D), b_f.reshape(1, _D))

    src = edge_index[0]
    dst = edge_index[1]
    pad_e = _EPAD - _E
    src_p = jnp.concatenate([src, jnp.full((pad_e,), _N, jnp.int32)])
    dst_p = jnp.concatenate([dst, jnp.zeros((pad_e,), jnp.int32)])
    src2 = jnp.stack([src_p, src_p + _NP]).reshape(2, 16 * _NBLK, _EB)
    dst2 = jnp.stack([dst_p, dst_p + _NP]).reshape(2, 16 * _NBLK, _EB)
    dstr = jnp.broadcast_to(dst_p.reshape(1, 16 * _NBLK, _EB),
                            (2, 16 * _NBLK, _EB))
    idx_packed = jnp.stack([src2, dst2, dstr], axis=2)

    sums = _edge_kernel(
        a_st.reshape(2 * _NP, _D), b_st.reshape(2 * _NP, _D), idx_packed)

    mupre = _k2(xiou, sums, U_iou, W_mu, W_lv,
                b_mu.reshape(1, _L), b_lv.reshape(1, _L))

    opid_p = jnp.pad(operation_id, (0, _NP - _N))
    head = _head_kernel(opid_p, mupre, op_table)
    return head[:_N, :_L], head[:_N, _L:]
